# baseline (speedup 1.0000x reference)
import jax
import jax.numpy as jnp
from jax import lax
from jax.experimental import pallas as pl
from jax.experimental.pallas import tpu as pltpu
from jax.experimental.pallas import tpu_sc as plsc

B_ = 4
S_ = 8192
D_ = 1024
NW = 32
RW = S_ // NW
CH = 8
NCH = RW // CH
UNROLL = 4


def _sc_body(x_hbm, p_hbm, o_hbm,
             xb0, xb1, pb0, pb1, ob0, ob1,
             sx0, sx1, sp0, sp1, so0, so1):
    wid = lax.axis_index("s") * 2 + lax.axis_index("c")
    row0 = wid * RW
    xbufs = (xb0, xb1)
    pbufs = (pb0, pb1)
    obufs = (ob0, ob1)
    sxs = (sx0, sx1)
    sps = (sp0, sp1)
    sos = (so0, so1)

    pltpu.async_copy(p_hbm.at[pl.ds(row0, CH)], pb0, sp0)
    pltpu.async_copy(x_hbm.at[0, pl.ds(row0, CH)], xb0, sx0)
    pltpu.async_copy(x_hbm.at[1, pl.ds(row0, CH)], xb1, sx1)

    @pl.loop(0, NCH, step=2)
    def _outer(c0):
        for pc in range(2):
            c = c0 + pc
            pltpu.make_async_copy(
                p_hbm.at[pl.ds(0, CH)], pbufs[pc], sps[pc]).wait()

            @pl.when(c + 1 < NCH)
            def _():
                pltpu.async_copy(
                    p_hbm.at[pl.ds(row0 + (c + 1) * CH, CH)],
                    pbufs[1 - pc], sps[1 - pc])

            for b in range(4):
                sl = b % 2
                pltpu.make_async_copy(
                    x_hbm.at[0, pl.ds(0, CH)], xbufs[sl], sxs[sl]).wait()
                if pc > 0 or b >= 2:
                    pltpu.make_async_copy(
                        obufs[sl], o_hbm.at[0, pl.ds(0, CH)], sos[sl]).wait()
                else:
                    @pl.when(c0 > 0)
                    def _():
                        pltpu.make_async_copy(
                            obufs[sl], o_hbm.at[0, pl.ds(0, CH)],
                            sos[sl]).wait()
                xb, pb, ob = xbufs[sl], pbufs[pc], obufs[sl]

                @plsc.parallel_loop(0, CH)
                def _compute(r):
                    for col in range(0, D_, 16):
                        sl16 = pl.ds(col, 16)
                        ob[r, sl16] = xb[r, sl16] + pb[r, sl16]

                pltpu.async_copy(
                    ob, o_hbm.at[b, pl.ds(row0 + c * CH, CH)], sos[sl])
                b2 = (b + 2) % 4
                c2 = c + (1 if b >= 2 else 0)

                @pl.when(c2 < NCH)
                def _():
                    pltpu.async_copy(
                        x_hbm.at[b2, pl.ds(row0 + c2 * CH, CH)],
                        xbufs[sl], sxs[sl])

    pltpu.make_async_copy(ob0, o_hbm.at[0, pl.ds(0, CH)], so0).wait()
    pltpu.make_async_copy(ob1, o_hbm.at[0, pl.ds(0, CH)], so1).wait()


_sc_add = pl.kernel(
    _sc_body,
    out_type=jax.ShapeDtypeStruct((B_, S_, D_), jnp.float32),
    mesh=plsc.VectorSubcoreMesh(core_axis_name="c", subcore_axis_name="s"),
    scratch_types=[pltpu.VMEM((CH, D_), jnp.float32)] * 6
    + [pltpu.SemaphoreType.DMA] * 6,
    compiler_params=pltpu.CompilerParams(use_tc_tiling_on_sc=True),
)


def kernel(x, pos_table):
    return _sc_add(x, pos_table)


# speedup vs baseline: 1.2345x; 1.2345x over previous
import jax
import jax.numpy as jnp
from jax import lax
from jax.experimental import pallas as pl
from jax.experimental.pallas import tpu as pltpu
from jax.experimental.pallas import tpu_sc as plsc

B_ = 4
S_ = 8192
D_ = 1024
NW = 32
RW = S_ // NW
CH = 16
NCH = RW // CH
UNROLL = 4


def _sc_body(x_hbm, p_hbm, o_hbm,
             xb0, xb1, pb0, pb1, ob0, ob1,
             sx0, sx1, sp0, sp1, so0, so1):
    wid = lax.axis_index("s") * 2 + lax.axis_index("c")
    row0 = wid * RW
    xbufs = (xb0, xb1)
    pbufs = (pb0, pb1)
    obufs = (ob0, ob1)
    sxs = (sx0, sx1)
    sps = (sp0, sp1)
    sos = (so0, so1)

    pltpu.async_copy(p_hbm.at[pl.ds(row0, CH)], pb0, sp0)
    pltpu.async_copy(x_hbm.at[0, pl.ds(row0, CH)], xb0, sx0)
    pltpu.async_copy(x_hbm.at[1, pl.ds(row0, CH)], xb1, sx1)

    @pl.loop(0, NCH, step=2)
    def _outer(c0):
        for pc in range(2):
            c = c0 + pc
            pltpu.make_async_copy(
                p_hbm.at[pl.ds(0, CH)], pbufs[pc], sps[pc]).wait()

            @pl.when(c + 1 < NCH)
            def _():
                pltpu.async_copy(
                    p_hbm.at[pl.ds(row0 + (c + 1) * CH, CH)],
                    pbufs[1 - pc], sps[1 - pc])

            for b in range(4):
                sl = b % 2
                pltpu.make_async_copy(
                    x_hbm.at[0, pl.ds(0, CH)], xbufs[sl], sxs[sl]).wait()
                if pc > 0 or b >= 2:
                    pltpu.make_async_copy(
                        obufs[sl], o_hbm.at[0, pl.ds(0, CH)], sos[sl]).wait()
                else:
                    @pl.when(c0 > 0)
                    def _():
                        pltpu.make_async_copy(
                            obufs[sl], o_hbm.at[0, pl.ds(0, CH)],
                            sos[sl]).wait()
                xb, pb, ob = xbufs[sl], pbufs[pc], obufs[sl]

                @plsc.parallel_loop(0, CH)
                def _compute(r):
                    for col in range(0, D_, 16):
                        sl16 = pl.ds(col, 16)
                        ob[r, sl16] = xb[r, sl16] + pb[r, sl16]

                pltpu.async_copy(
                    ob, o_hbm.at[b, pl.ds(row0 + c * CH, CH)], sos[sl])
                b2 = (b + 2) % 4
                c2 = c + (1 if b >= 2 else 0)

                @pl.when(c2 < NCH)
                def _():
                    pltpu.async_copy(
                        x_hbm.at[b2, pl.ds(row0 + c2 * CH, CH)],
                        xbufs[sl], sxs[sl])

    pltpu.make_async_copy(ob0, o_hbm.at[0, pl.ds(0, CH)], so0).wait()
    pltpu.make_async_copy(ob1, o_hbm.at[0, pl.ds(0, CH)], so1).wait()


_sc_add = pl.kernel(
    _sc_body,
    out_type=jax.ShapeDtypeStruct((B_, S_, D_), jnp.float32),
    mesh=plsc.VectorSubcoreMesh(core_axis_name="c", subcore_axis_name="s"),
    scratch_types=[pltpu.VMEM((CH, D_), jnp.float32)] * 6
    + [pltpu.SemaphoreType.DMA] * 6,
    compiler_params=pltpu.CompilerParams(use_tc_tiling_on_sc=True),
)


def kernel(x, pos_table):
    return _sc_add(x, pos_table)


# speedup vs baseline: 1.2375x; 1.0024x over previous
import jax
import jax.numpy as jnp
from jax import lax
from jax.experimental import pallas as pl
from jax.experimental.pallas import tpu as pltpu
from jax.experimental.pallas import tpu_sc as plsc

B_ = 4
S_ = 8192
D_ = 1024
NW = 32
RW = S_ // NW
CH = 16
NCH = RW // CH


def _sc_body(x_hbm, p_hbm, o_hbm,
             xb0, xb1, pb0, pb1, ob0, ob1,
             sx0, sx1, sp0, sp1, so0, so1):
    wid = lax.axis_index("s") * 2 + lax.axis_index("c")
    row0 = wid * RW
    xbufs = (xb0, xb1)
    pbufs = (pb0, pb1)
    obufs = (ob0, ob1)
    sxs = (sx0, sx1)
    sps = (sp0, sp1)
    sos = (so0, so1)

    pltpu.async_copy(p_hbm.at[pl.ds(row0, CH)], pb0, sp0)
    pltpu.async_copy(x_hbm.at[0, pl.ds(row0, CH)], xb0, sx0)
    pltpu.async_copy(x_hbm.at[1, pl.ds(row0, CH)], xb1, sx1)

    @pl.loop(0, NCH, step=2)
    def _outer(c0):
        for pc in range(2):
            c = c0 + pc
            pltpu.make_async_copy(
                p_hbm.at[pl.ds(0, CH)], pbufs[pc], sps[pc]).wait()

            @pl.when(c + 1 < NCH)
            def _():
                pltpu.async_copy(
                    p_hbm.at[pl.ds(row0 + (c + 1) * CH, CH)],
                    pbufs[1 - pc], sps[1 - pc])

            for b in range(4):
                sl = b % 2
                pltpu.make_async_copy(
                    x_hbm.at[0, pl.ds(0, CH)], xbufs[sl], sxs[sl]).wait()
                if pc > 0 or b >= 2:
                    pltpu.make_async_copy(
                        obufs[sl], o_hbm.at[0, pl.ds(0, CH)], sos[sl]).wait()
                else:
                    @pl.when(c0 > 0)
                    def _():
                        pltpu.make_async_copy(
                            obufs[sl], o_hbm.at[0, pl.ds(0, CH)],
                            sos[sl]).wait()
                xb, pb, ob = xbufs[sl], pbufs[pc], obufs[sl]

                @plsc.parallel_loop(0, CH)
                def _compute(r):
                    for col in range(0, D_, 16):
                        sl16 = pl.ds(col, 16)
                        ob[r, sl16] = xb[r, sl16] + pb[r, sl16]

                pltpu.async_copy(
                    ob, o_hbm.at[b, pl.ds(row0 + c * CH, CH)], sos[sl])
                b2 = (b + 2) % 4
                c2 = c + (1 if b >= 2 else 0)

                @pl.when(c2 < NCH)
                def _():
                    pltpu.async_copy(
                        x_hbm.at[b2, pl.ds(row0 + c2 * CH, CH)],
                        xbufs[sl], sxs[sl])

    pltpu.make_async_copy(ob0, o_hbm.at[0, pl.ds(0, CH)], so0).wait()
    pltpu.make_async_copy(ob1, o_hbm.at[0, pl.ds(0, CH)], so1).wait()


_sc_add = pl.kernel(
    _sc_body,
    out_type=jax.ShapeDtypeStruct((B_, S_, D_), jnp.float32),
    mesh=plsc.VectorSubcoreMesh(core_axis_name="c", subcore_axis_name="s"),
    scratch_types=[pltpu.VMEM((CH, D_), jnp.float32)] * 6
    + [pltpu.SemaphoreType.DMA] * 6,
    compiler_params=pltpu.CompilerParams(use_tc_tiling_on_sc=True),
)


def kernel(x, pos_table):
    return _sc_add(x, pos_table)
